# Initial kernel scaffold; baseline (speedup 1.0000x reference)
import jax
import jax.numpy as jnp
from jax.experimental import pallas as pl


def kernel(seq_index, item_indices, target_index, seq_emb, item_emb, out_emb, neg_indices):
    raise NotImplementedError("write your pallas kernel here")


# speedup vs baseline: 1.9128x; 1.9128x over previous
import functools

import jax
import jax.numpy as jnp
from jax import lax
from jax.experimental import pallas as pl
from jax.experimental.pallas import tpu as pltpu
from jax.experimental.pallas import tpu_sc as plsc

D = 64
LANES = 16
NV = D // LANES


def _make_sc_logits(B, W, NEG):
    info = plsc.get_sparse_core_info()
    NC, NS = info.num_cores, info.num_subcores
    NW = NC * NS
    BW = B // NW
    CH = 32
    NCH = BW // CH

    mesh = plsc.VectorSubcoreMesh(core_axis_name="c", subcore_axis_name="s")
    inv_ws = 1.0 / (W + 1)

    @functools.partial(
        pl.kernel,
        mesh=mesh,
        compiler_params=pltpu.CompilerParams(
            needs_layout_passes=False, use_tc_tiling_on_sc=False),
        out_type=[
            jax.ShapeDtypeStruct((B,), jnp.float32),
            jax.ShapeDtypeStruct((NEG, B), jnp.float32),
        ],
        scratch_types=[
            pltpu.VMEM((BW,), jnp.int32),
            pltpu.VMEM((W, BW), jnp.int32),
            pltpu.VMEM((BW,), jnp.int32),
            pltpu.VMEM((NEG, BW), jnp.int32),
            pltpu.VMEM((CH, D), jnp.float32),
            pltpu.VMEM((NEG, CH, D), jnp.float32),
            pltpu.VMEM((D, CH), jnp.float32),
            pltpu.VMEM((BW,), jnp.float32),
            pltpu.VMEM((NEG, BW), jnp.float32),
            pltpu.SemaphoreType.DMA,
        ],
    )
    def sc_logits(seq_idx_h, item_idx_h, tgt_idx_h, neg_idx_h,
                  seq_emb_h, item_emb_h, out_emb_h,
                  xpos_h, xneg_h,
                  seqi_v, itemi_v, tgti_v, negi_v,
                  srow_v, rows_v, p_v, xpos_v, xneg_v, sem):
        wid = lax.axis_index("s") * NC + lax.axis_index("c")
        base = wid * BW

        pltpu.sync_copy(seq_idx_h.at[pl.ds(base, BW)], seqi_v)
        pltpu.sync_copy(tgt_idx_h.at[pl.ds(base, BW)], tgti_v)
        pltpu.sync_copy(item_idx_h.at[:, pl.ds(base, BW)], itemi_v)
        pltpu.sync_copy(neg_idx_h.at[:, pl.ds(base, BW)], negi_v)

        for c in range(NCH):
            co = c * CH

            cps = [pltpu.async_copy(
                seq_emb_h.at[seqi_v.at[pl.ds(co, CH)]], srow_v, sem)]
            for w in range(W):
                cps.append(pltpu.async_copy(
                    item_emb_h.at[itemi_v.at[w, pl.ds(co, CH)]],
                    rows_v.at[w], sem))
            for cp in cps:
                cp.wait()

            lane = lax.iota(jnp.int32, LANES)
            b16s = [jnp.full((LANES,), g * LANES, jnp.int32) + lane
                    for g in range(CH // LANES)]

            def body_a(d, carry):
                dcol = jnp.full((LANES,), d, jnp.int32)
                for g, b16 in enumerate(b16s):
                    acc = plsc.load_gather(srow_v, [b16, dcol])
                    for w in range(W):
                        acc = acc + plsc.load_gather(
                            rows_v, [jnp.full((LANES,), w, jnp.int32), b16, dcol])
                    p_v[d, pl.ds(g * LANES, LANES)] = acc * inv_ws
                return carry

            lax.fori_loop(0, D, body_a, 0)

            cps = [pltpu.async_copy(
                out_emb_h.at[tgti_v.at[pl.ds(co, CH)]], srow_v, sem)]
            for n in range(NEG):
                cps.append(pltpu.async_copy(
                    out_emb_h.at[negi_v.at[n, pl.ds(co, CH)]],
                    rows_v.at[n], sem))
            for cp in cps:
                cp.wait()

            zero = jnp.zeros((LANES,), jnp.float32)
            for g, b16 in enumerate(b16s):

                def body_b(d, accs):
                    dcol = jnp.full((LANES,), d, jnp.int32)
                    pt = p_v[d, pl.ds(g * LANES, LANES)]
                    out = [accs[0] + pt * plsc.load_gather(srow_v, [b16, dcol])]
                    for n in range(NEG):
                        out.append(accs[n + 1] + pt * plsc.load_gather(
                            rows_v, [jnp.full((LANES,), n, jnp.int32), b16, dcol]))
                    return tuple(out)

                accs = lax.fori_loop(0, D, body_b, (zero,) * (NEG + 1))
                xpos_v[pl.ds(co + g * LANES, LANES)] = accs[0]
                for n in range(NEG):
                    xneg_v[n, pl.ds(co + g * LANES, LANES)] = accs[n + 1]

        pltpu.sync_copy(xpos_v, xpos_h.at[pl.ds(base, BW)])
        pltpu.sync_copy(xneg_v, xneg_h.at[:, pl.ds(base, BW)])

    return sc_logits


def _loss_body(neg_n, xp_ref, xn_ref, o_ref):
    eps = 1e-7
    pos_c = jnp.clip(jax.nn.sigmoid(xp_ref[...]), eps, 1.0 - eps)
    neg_c = jnp.clip(jax.nn.sigmoid(xn_ref[...]), eps, 1.0 - eps)
    loss_pos = -jnp.mean(jnp.log(pos_c))
    loss_neg = -jnp.mean(jnp.log(1.0 - neg_c))
    o_ref[...] = ((loss_pos + loss_neg / neg_n) / 2.0).reshape(1, 1)


def kernel(seq_index, item_indices, target_index, seq_emb, item_emb, out_emb, neg_indices):
    B, W = item_indices.shape
    NEG = neg_indices.shape[1]

    sc_logits = _make_sc_logits(B, W, NEG)
    x_pos, x_neg = sc_logits(
        seq_index.astype(jnp.int32),
        item_indices.T.astype(jnp.int32),
        target_index.astype(jnp.int32),
        neg_indices.T.astype(jnp.int32),
        seq_emb, item_emb, out_emb,
    )

    loss = pl.pallas_call(
        functools.partial(_loss_body, NEG),
        out_shape=jax.ShapeDtypeStruct((1, 1), jnp.float32),
    )(x_pos.reshape(B // 128, 128), x_neg)

    return loss[0, 0]
